# Initial kernel scaffold; baseline (speedup 1.0000x reference)
import jax
import jax.numpy as jnp
from jax.experimental import pallas as pl


def kernel(x, edge_index, W1, b1, Wg1, bg1, Wg2, bg2, W2, b2):
    raise NotImplementedError("write your pallas kernel here")


# speedup vs baseline: 3.5934x; 3.5934x over previous
import functools

import jax
import jax.numpy as jnp
from jax import lax
from jax.experimental import pallas as pl
from jax.experimental.pallas import tpu as pltpu
from jax.experimental.pallas import tpu_sc as plsc

N = 10000
D = 128
E = 320000
NPAD = 10240
NC = 2
NS = 16
NT = NC * NS
K = 64
NB = 160
CH = 16
NCH = NB // CH
EP = NT * NB * K
RPT = NPAD // NS


def _make_sc_agg(with_deg: bool):
    mesh = plsc.VectorSubcoreMesh(
        core_axis_name="c", subcore_axis_name="s", num_cores=NC,
        num_subcores=NS)

    out_type = [jax.ShapeDtypeStruct((NC, NPAD, D), jnp.float32)]
    if with_deg:
        out_type.append(
            jax.ShapeDtypeStruct((NC, NPAD // D, D), jnp.float32))

    scratch = dict(
        acc_sh=pltpu.VMEM_SHARED((NPAD, D), jnp.float32),
        src_v=pltpu.VMEM((CH, K), jnp.int32),
        dst_v=pltpu.VMEM((CH, K), jnp.int32),
        rows_v=pltpu.VMEM((2, K, D), jnp.float32),
        tag_v=pltpu.VMEM((NPAD,), jnp.int32),
        idx_v=pltpu.VMEM((K,), jnp.int32),
        sem_a=pltpu.SemaphoreType.DMA,
        sem_b=pltpu.SemaphoreType.DMA,
    )
    if with_deg:
        scratch["deg_sh"] = pltpu.VMEM_SHARED((NPAD // D, D), jnp.float32)
        scratch["degt_v"] = pltpu.VMEM((NPAD // D, D), jnp.float32)
        scratch["iden_v"] = pltpu.VMEM((NPAD // D,), jnp.int32)

    @functools.partial(
        pl.kernel, out_type=tuple(out_type), mesh=mesh,
        compiler_params=pltpu.CompilerParams(
            needs_layout_passes=False, internal_scratch_in_bytes=4096),
        scratch_types=scratch)
    def sc_agg(h_hbm, src_hbm, dst_hbm, *args, acc_sh, src_v, dst_v,
               rows_v, tag_v, idx_v, sem_a, sem_b, deg_sh=None, degt_v=None,
               iden_v=None):
        if with_deg:
            out_hbm, deg_hbm = args[0], args[1]
        else:
            out_hbm = args[0]

        c = lax.axis_index("c")
        s = lax.axis_index("s")
        t = c * NS + s
        lanes = lax.iota(jnp.int32, 16)
        DR = NPAD // D

        def _zrow(i, _):
            for kk in range(D // 16):
                rows_v[0, i, pl.ds(kk * 16, 16)] = jnp.zeros(
                    (16,), jnp.float32)
            return 0
        lax.fori_loop(0, K, _zrow, 0)
        base = s * RPT
        for kc in range(RPT // K):
            pltpu.sync_copy(rows_v.at[0], acc_sh.at[pl.ds(base + kc * K, K)])
        if with_deg:
            def _zdeg(i, _):
                for kk in range(D // 16):
                    degt_v[i, pl.ds(kk * 16, 16)] = jnp.zeros(
                        (16,), jnp.float32)
                return 0
            lax.fori_loop(0, DR, _zdeg, 0)
            for r in range(DR // 16):
                iden_v[pl.ds(r * 16, 16)] = lanes + r * 16
            @pl.when(s < DR // 8)
            def _():
                pltpu.sync_copy(degt_v.at[pl.ds(s * 8, 8)],
                                deg_sh.at[pl.ds(s * 8, 8)])
        plsc.subcore_barrier()

        NV = K // 16
        sems = (sem_a, sem_b)

        def _batch(j, buf):
            dvs = [dst_v[j, pl.ds(r * 16, 16)] for r in range(NV)]
            lids = [lanes + r * 16 for r in range(NV)]

            def _cond(carry):
                return carry[0] > 0

            def _round(carry):
                cnt = carry[0]
                acts = list(carry[1:])
                for r in range(NV):
                    plsc.store_scatter(tag_v, [dvs[r]], lids[r],
                                       mask=acts[r] != 0)
                new_acts = []
                nwin = jnp.int32(0)
                for r in range(NV):
                    tb = plsc.load_gather(tag_v, [dvs[r]])
                    win = (acts[r] != 0) & (tb == lids[r])
                    idx_v[pl.ds(r * 16, 16)] = jnp.where(
                        win, dvs[r], N + lids[r])
                    if with_deg:
                        plsc.addupdate_scatter(
                            degt_v,
                            [lax.shift_right_logical(dvs[r], 7),
                             lax.bitwise_and(dvs[r], 127)],
                            jnp.ones((16,), jnp.float32), mask=win)
                    new_acts.append(jnp.where(win, 0, acts[r]))
                    nwin = nwin + jnp.sum(jnp.where(win, 1, 0))
                pltpu.sync_copy(rows_v.at[buf], acc_sh.at[idx_v], add=True)
                return (cnt - nwin, *new_acts)

            ones_i = jnp.ones((16,), jnp.int32)
            lax.while_loop(_cond, _round,
                           (jnp.int32(K), *([ones_i] * NV)))

        def _chunk(ci, _):
            pltpu.sync_copy(src_hbm.at[t, pl.ds(ci * CH, CH)], src_v)
            pltpu.sync_copy(dst_hbm.at[t, pl.ds(ci * CH, CH)], dst_v)
            pending = pltpu.async_copy(
                h_hbm.at[src_v.at[0]], rows_v.at[0], sems[0])
            for j in range(CH):
                buf = j % 2
                nxt = pending
                if j + 1 < CH:
                    nxt = pltpu.async_copy(
                        h_hbm.at[src_v.at[j + 1]], rows_v.at[1 - buf],
                        sems[1 - buf])
                pending.wait()
                _batch(j, buf)
                pending = nxt
            return 0
        lax.fori_loop(0, NCH, _chunk, 0)
        plsc.subcore_barrier()

        if with_deg:
            pltpu.sync_copy(degt_v, deg_sh.at[iden_v], add=True)
            plsc.subcore_barrier()

        for kc in range(RPT // K):
            r0 = base + kc * K
            pltpu.sync_copy(acc_sh.at[pl.ds(r0, K)], rows_v.at[0])
            pltpu.sync_copy(rows_v.at[0], out_hbm.at[c, pl.ds(r0, K)])
        if with_deg:
            @pl.when(s < DR // 8)
            def _():
                pltpu.sync_copy(deg_sh.at[pl.ds(s * 8, 8)],
                                degt_v.at[pl.ds(0, 8)])
                pltpu.sync_copy(degt_v.at[pl.ds(0, 8)],
                                deg_hbm.at[c, pl.ds(s * 8, 8)])

    return sc_agg


_make_sc_agg = functools.cache(_make_sc_agg)

_BLK = 1024
_BLK3 = 1000


def _tc1_body(x_ref, w_ref, b_ref, o_ref):
    h = jnp.dot(x_ref[...], w_ref[...],
                preferred_element_type=jnp.float32) + b_ref[...]
    nrm = jnp.sqrt(jnp.sum(h * h, axis=-1, keepdims=True))
    o_ref[...] = h / jnp.maximum(nrm, 1e-20)


def _tc2_body(p_ref, d_ref, w_ref, b_ref, o_ref):
    agg = p_ref[0] + p_ref[1]
    deg = jnp.maximum(d_ref[0] + d_ref[1], 1.0)
    h1 = jnp.dot(agg / deg, w_ref[...],
                 preferred_element_type=jnp.float32) + b_ref[...]
    o_ref[...] = jnp.maximum(h1, 0.0)


def _tc3_body(p_ref, d_ref, x_ref, wg2_ref, bg2_ref, w2_ref, b2_ref, o_ref):
    agg = p_ref[0] + p_ref[1]
    deg = jnp.maximum(d_ref[0] + d_ref[1], 1.0)
    h2 = jnp.dot(agg / deg, wg2_ref[...],
                 preferred_element_type=jnp.float32) + bg2_ref[...]
    recon = jnp.dot(h2, w2_ref[...],
                    preferred_element_type=jnp.float32) + b2_ref[...]
    r = x_ref[...] - recon
    o_ref[...] = jnp.sum(r * r, axis=-1).reshape(1, 1, _BLK3)


def kernel(x, edge_index, W1, b1, Wg1, bg1, Wg2, bg2, W2, b2):
    xp = jnp.pad(x, ((0, NPAD - N), (0, 0)))
    src = edge_index[0]
    dst = edge_index[1]
    pad_e = EP - E
    srcm = jnp.concatenate(
        [src, jnp.zeros((pad_e,), jnp.int32)]).reshape(NT, NB, K)
    dstm = jnp.concatenate(
        [dst, N + (jnp.arange(pad_e, dtype=jnp.int32) % K)]
    ).reshape(NT, NB, K)

    h = pl.pallas_call(
        _tc1_body,
        out_shape=jax.ShapeDtypeStruct((NPAD, D), jnp.float32),
        grid=(NPAD // _BLK,),
        in_specs=[
            pl.BlockSpec((_BLK, D), lambda i: (i, 0)),
            pl.BlockSpec((D, D), lambda i: (0, 0)),
            pl.BlockSpec((1, D), lambda i: (0, 0)),
        ],
        out_specs=pl.BlockSpec((_BLK, D), lambda i: (i, 0)),
    )(xp, W1, b1.reshape(1, D))

    p1, dg = _make_sc_agg(True)(h, srcm, dstm)
    dg = dg.reshape(NC, NPAD, 1)

    h1 = pl.pallas_call(
        _tc2_body,
        out_shape=jax.ShapeDtypeStruct((NPAD, D), jnp.float32),
        grid=(NPAD // _BLK,),
        in_specs=[
            pl.BlockSpec((NC, _BLK, D), lambda i: (0, i, 0)),
            pl.BlockSpec((NC, _BLK, 1), lambda i: (0, i, 0)),
            pl.BlockSpec((D, D), lambda i: (0, 0)),
            pl.BlockSpec((1, D), lambda i: (0, 0)),
        ],
        out_specs=pl.BlockSpec((_BLK, D), lambda i: (i, 0)),
    )(p1, dg, Wg1, bg1.reshape(1, D))

    (p2,) = _make_sc_agg(False)(h1, srcm, dstm)

    out = pl.pallas_call(
        _tc3_body,
        out_shape=jax.ShapeDtypeStruct((N // _BLK3, 1, _BLK3), jnp.float32),
        grid=(N // _BLK3,),
        in_specs=[
            pl.BlockSpec((NC, _BLK3, D), lambda i: (0, i, 0)),
            pl.BlockSpec((NC, _BLK3, 1), lambda i: (0, i, 0)),
            pl.BlockSpec((_BLK3, D), lambda i: (i, 0)),
            pl.BlockSpec((D, D), lambda i: (0, 0)),
            pl.BlockSpec((1, D), lambda i: (0, 0)),
            pl.BlockSpec((D, D), lambda i: (0, 0)),
            pl.BlockSpec((1, D), lambda i: (0, 0)),
        ],
        out_specs=pl.BlockSpec((1, 1, _BLK3), lambda i: (i, 0, 0)),
    )(p2, dg, x, Wg2, bg2.reshape(1, D), W2, b2.reshape(1, D))

    return out.reshape(N)
